# baseline (speedup 1.0000x reference)
import functools

import jax
import jax.numpy as jnp
from jax import lax
from jax.experimental import pallas as pl
from jax.experimental.pallas import tpu as pltpu
from jax.experimental.pallas import tpu_sc as plsc

BATCH = 16384
NROWS = 1000000
D = 32
LANES = 16
NC = 2
NS = 16
NW = NC * NS
BPW = BATCH // NW
SB = 4
NSB = BPW // SB
SUPER = 16
NSUPER = BPW // SUPER


@functools.partial(
    pl.kernel,
    out_type=jax.ShapeDtypeStruct((BATCH,), jnp.float32),
    mesh=plsc.VectorSubcoreMesh(core_axis_name="c", subcore_axis_name="s"),
    compiler_params=pltpu.CompilerParams(
        needs_layout_passes=False, disable_bounds_checks=True
    ),
    scratch_types=[
        pltpu.VMEM((BPW + SUPER,), jnp.int32),
        pltpu.VMEM((BPW + SUPER,), jnp.int32),
        pltpu.VMEM((2, SB, 2, 4, 8, 128), jnp.float32),
        pltpu.VMEM((BPW,), jnp.float32),
        pltpu.SemaphoreType.DMA,
        pltpu.SemaphoreType.DMA,
    ],
)
def _neumf_sc(users_hbm, items_hbm, utv_hbm, itv_hbm, out_hbm,
              idx_u, idx_i, ring, out_v, sem_a, sem_b):
    wid = lax.axis_index("s") * NC + lax.axis_index("c")
    base = wid * BPW

    pltpu.sync_copy(users_hbm.at[pl.ds(base, BPW)], idx_u.at[pl.ds(0, BPW)])
    pltpu.sync_copy(items_hbm.at[pl.ds(base, BPW)], idx_i.at[pl.ds(0, BPW)])
    idx_u[pl.ds(BPW, SUPER)] = jnp.zeros((SUPER,), jnp.int32)
    idx_i[pl.ds(BPW, SUPER)] = jnp.zeros((SUPER,), jnp.int32)

    lanes = lax.iota(jnp.int32, LANES)
    db_lo = lanes // 8
    db_hi = db_lo + 2
    d8 = lanes % 8
    is_last = lanes == LANES - 1

    def issue(rv_u, rv_i, k0, slot, sem):
        for k in range(SB):
            ru = rv_u[k0 + k]
            ri = rv_i[k0 + k]
            tu = pl.multiple_of((ru // 128) * 128, 128)
            ti = pl.multiple_of((ri // 128) * 128, 128)
            for db in range(4):
                pltpu.async_copy(
                    utv_hbm.at[pl.ds(db * 8, 8), pl.ds(tu, 128)],
                    ring.at[slot, k, 0, db], sem)
                pltpu.async_copy(
                    itv_hbm.at[pl.ds(db * 8, 8), pl.ds(ti, 128)],
                    ring.at[slot, k, 1, db], sem)

    def drain(sem):
        for _ in range(SB * 8):
            pltpu.make_async_copy(
                utv_hbm.at[pl.ds(0, 8), pl.ds(0, 128)],
                ring.at[0, 0, 0, 0], sem).wait()

    def compute(rv_u, rv_i, k0, slot, nb, j):
        for k in range(SB):
            lu = jnp.full((LANES,), rv_u[k0 + k] % 128, jnp.int32)
            li = jnp.full((LANES,), rv_i[k0 + k] % 128, jnp.int32)
            slot_v = jnp.full((LANES,), slot, jnp.int32)
            k_v = jnp.full((LANES,), k, jnp.int32)
            t0 = jnp.zeros((LANES,), jnp.int32)
            t1 = jnp.full((LANES,), 1, jnp.int32)
            u_lo = plsc.load_gather(ring, [slot_v, k_v, t0, db_lo, d8, lu])
            u_hi = plsc.load_gather(ring, [slot_v, k_v, t0, db_hi, d8, lu])
            i_lo = plsc.load_gather(ring, [slot_v, k_v, t1, db_lo, d8, li])
            i_hi = plsc.load_gather(ring, [slot_v, k_v, t1, db_hi, d8, li])
            acc = u_lo * i_lo + u_hi * i_hi
            tot = jnp.cumsum(acc)
            pos = nb * SUPER + j * SB + k
            plsc.store_scatter(out_v, [jnp.full((LANES,), pos, jnp.int32)],
                               tot, mask=is_last)

    rv_u0 = idx_u[pl.ds(0, LANES)]
    rv_i0 = idx_i[pl.ds(0, LANES)]
    issue(rv_u0, rv_i0, 0, 0, sem_a)

    def super_batch(nb, carry):
        off = pl.multiple_of(nb * SUPER, SUPER)
        rv_u = idx_u[pl.ds(off, LANES)]
        rv_i = idx_i[pl.ds(off, LANES)]
        off_n = pl.multiple_of(nb * SUPER + SUPER, SUPER)
        rv_un = idx_u[pl.ds(off_n, LANES)]
        rv_in = idx_i[pl.ds(off_n, LANES)]
        for j in range(SUPER // SB):
            slot, sem = j % 2, (sem_a, sem_b)[j % 2]
            slot_n, sem_n = (j + 1) % 2, (sem_a, sem_b)[(j + 1) % 2]
            if j + 1 < SUPER // SB:
                issue(rv_u, rv_i, (j + 1) * SB, slot_n, sem_n)
            else:
                issue(rv_un, rv_in, 0, slot_n, sem_n)
            drain(sem)
            compute(rv_u, rv_i, j * SB, slot, nb, j)
        return carry

    lax.fori_loop(0, NSUPER, super_batch, 0)
    drain(sem_a)

    pltpu.sync_copy(out_v, out_hbm.at[pl.ds(base, BPW)])


def kernel(users, items, user_table, item_table):
    return _neumf_sc(users.astype(jnp.int32), items.astype(jnp.int32),
                     user_table.T, item_table.T)


# speedup vs baseline: 1.0619x; 1.0619x over previous
import functools

import jax
import jax.numpy as jnp
from jax import lax
from jax.experimental import pallas as pl
from jax.experimental.pallas import tpu as pltpu
from jax.experimental.pallas import tpu_sc as plsc

BATCH = 16384
NROWS = 1000000
D = 32
LANES = 16
NC = 2
NS = 16
NW = NC * NS
BPW = BATCH // NW
SB = 2
NSLOT = 4
DEPTH = 3
SUPER = 16
NSUPER = BPW // SUPER
JPS = SUPER // SB


@functools.partial(
    pl.kernel,
    out_type=jax.ShapeDtypeStruct((BATCH,), jnp.float32),
    mesh=plsc.VectorSubcoreMesh(core_axis_name="c", subcore_axis_name="s"),
    compiler_params=pltpu.CompilerParams(
        needs_layout_passes=False, disable_bounds_checks=True
    ),
    scratch_types=[
        pltpu.VMEM((BPW + SUPER,), jnp.int32),
        pltpu.VMEM((BPW + SUPER,), jnp.int32),
        pltpu.VMEM((NSLOT, SB, 2, 4, 8, 128), jnp.float32),
        pltpu.VMEM((BPW,), jnp.float32),
        pltpu.SemaphoreType.DMA,
        pltpu.SemaphoreType.DMA,
        pltpu.SemaphoreType.DMA,
        pltpu.SemaphoreType.DMA,
    ],
)
def _neumf_sc(users_hbm, items_hbm, utv_hbm, itv_hbm, out_hbm,
              idx_u, idx_i, ring, out_v, sem0, sem1, sem2, sem3):
    sems = (sem0, sem1, sem2, sem3)
    wid = lax.axis_index("s") * NC + lax.axis_index("c")
    base = wid * BPW

    pltpu.sync_copy(users_hbm.at[pl.ds(base, BPW)], idx_u.at[pl.ds(0, BPW)])
    pltpu.sync_copy(items_hbm.at[pl.ds(base, BPW)], idx_i.at[pl.ds(0, BPW)])
    idx_u[pl.ds(BPW, SUPER)] = jnp.zeros((SUPER,), jnp.int32)
    idx_i[pl.ds(BPW, SUPER)] = jnp.zeros((SUPER,), jnp.int32)

    lanes = lax.iota(jnp.int32, LANES)
    db_lo = lanes // 8
    db_hi = db_lo + 2
    d8 = lanes % 8
    is_last = lanes == LANES - 1

    def issue(rv_u, rv_i, k0, slot, sem):
        for k in range(SB):
            ru = rv_u[k0 + k]
            ri = rv_i[k0 + k]
            tu = pl.multiple_of((ru // 128) * 128, 128)
            ti = pl.multiple_of((ri // 128) * 128, 128)
            for db in range(4):
                pltpu.async_copy(
                    utv_hbm.at[pl.ds(db * 8, 8), pl.ds(tu, 128)],
                    ring.at[slot, k, 0, db], sem)
                pltpu.async_copy(
                    itv_hbm.at[pl.ds(db * 8, 8), pl.ds(ti, 128)],
                    ring.at[slot, k, 1, db], sem)

    def drain(sem):
        for _ in range(SB * 8):
            pltpu.make_async_copy(
                utv_hbm.at[pl.ds(0, 8), pl.ds(0, 128)],
                ring.at[0, 0, 0, 0], sem).wait()

    def compute(rv_u, rv_i, k0, slot, nb, j):
        for k in range(SB):
            lu = jnp.full((LANES,), rv_u[k0 + k] % 128, jnp.int32)
            li = jnp.full((LANES,), rv_i[k0 + k] % 128, jnp.int32)
            slot_v = jnp.full((LANES,), slot, jnp.int32)
            k_v = jnp.full((LANES,), k, jnp.int32)
            t0 = jnp.zeros((LANES,), jnp.int32)
            t1 = jnp.full((LANES,), 1, jnp.int32)
            u_lo = plsc.load_gather(ring, [slot_v, k_v, t0, db_lo, d8, lu])
            u_hi = plsc.load_gather(ring, [slot_v, k_v, t0, db_hi, d8, lu])
            i_lo = plsc.load_gather(ring, [slot_v, k_v, t1, db_lo, d8, li])
            i_hi = plsc.load_gather(ring, [slot_v, k_v, t1, db_hi, d8, li])
            acc = u_lo * i_lo + u_hi * i_hi
            tot = jnp.cumsum(acc)
            pos = nb * SUPER + j * SB + k
            plsc.store_scatter(out_v, [jnp.full((LANES,), pos, jnp.int32)],
                               tot, mask=is_last)

    rv_u0 = idx_u[pl.ds(0, LANES)]
    rv_i0 = idx_i[pl.ds(0, LANES)]
    for sb in range(DEPTH):
        issue(rv_u0, rv_i0, sb * SB, sb, sems[sb])

    def super_batch(nb, carry):
        off = pl.multiple_of(nb * SUPER, SUPER)
        rv_u = idx_u[pl.ds(off, LANES)]
        rv_i = idx_i[pl.ds(off, LANES)]
        off_n = pl.multiple_of(nb * SUPER + SUPER, SUPER)
        rv_un = idx_u[pl.ds(off_n, LANES)]
        rv_in = idx_i[pl.ds(off_n, LANES)]
        for j in range(JPS):
            slot = j % NSLOT
            jn = j + DEPTH
            slot_n = jn % NSLOT
            if jn < JPS:
                issue(rv_u, rv_i, jn * SB, slot_n, sems[slot_n])
            else:
                issue(rv_un, rv_in, (jn - JPS) * SB, slot_n, sems[slot_n])
            drain(sems[slot])
            compute(rv_u, rv_i, j * SB, slot, nb, j)
        return carry

    lax.fori_loop(0, NSUPER, super_batch, 0)
    for sb in range(DEPTH):
        drain(sems[sb % NSLOT])

    pltpu.sync_copy(out_v, out_hbm.at[pl.ds(base, BPW)])


def kernel(users, items, user_table, item_table):
    return _neumf_sc(users.astype(jnp.int32), items.astype(jnp.int32),
                     user_table.T, item_table.T)


# speedup vs baseline: 1.1052x; 1.0408x over previous
import functools

import jax
import jax.numpy as jnp
from jax import lax
from jax.experimental import pallas as pl
from jax.experimental.pallas import tpu as pltpu
from jax.experimental.pallas import tpu_sc as plsc

BATCH = 16384
NROWS = 1000000
D = 32
LANES = 16
NC = 2
NS = 16
NW = NC * NS
BPW = BATCH // NW
SB = 1
NSLOT = 8
DEPTH = 6
SUPER = 16
NSUPER = BPW // SUPER
JPS = SUPER // SB


@functools.partial(
    pl.kernel,
    out_type=jax.ShapeDtypeStruct((BATCH,), jnp.float32),
    mesh=plsc.VectorSubcoreMesh(core_axis_name="c", subcore_axis_name="s"),
    compiler_params=pltpu.CompilerParams(
        needs_layout_passes=False, disable_bounds_checks=True
    ),
    scratch_types=[
        pltpu.VMEM((BPW + SUPER,), jnp.int32),
        pltpu.VMEM((BPW + SUPER,), jnp.int32),
        pltpu.VMEM((NSLOT, SB, 2, 4, 8, 128), jnp.float32),
        pltpu.VMEM((BPW,), jnp.float32),
        pltpu.SemaphoreType.DMA,
        pltpu.SemaphoreType.DMA,
        pltpu.SemaphoreType.DMA,
        pltpu.SemaphoreType.DMA,
        pltpu.SemaphoreType.DMA,
        pltpu.SemaphoreType.DMA,
        pltpu.SemaphoreType.DMA,
        pltpu.SemaphoreType.DMA,
    ],
)
def _neumf_sc(users_hbm, items_hbm, utv_hbm, itv_hbm, out_hbm,
              idx_u, idx_i, ring, out_v,
              sem0, sem1, sem2, sem3, sem4, sem5, sem6, sem7):
    sems = (sem0, sem1, sem2, sem3, sem4, sem5, sem6, sem7)
    wid = lax.axis_index("s") * NC + lax.axis_index("c")
    base = wid * BPW

    pltpu.sync_copy(users_hbm.at[pl.ds(base, BPW)], idx_u.at[pl.ds(0, BPW)])
    pltpu.sync_copy(items_hbm.at[pl.ds(base, BPW)], idx_i.at[pl.ds(0, BPW)])
    idx_u[pl.ds(BPW, SUPER)] = jnp.zeros((SUPER,), jnp.int32)
    idx_i[pl.ds(BPW, SUPER)] = jnp.zeros((SUPER,), jnp.int32)

    lanes = lax.iota(jnp.int32, LANES)
    db_lo = lanes // 8
    db_hi = db_lo + 2
    d8 = lanes % 8
    is_last = lanes == LANES - 1

    def issue(rv_u, rv_i, k0, slot, sem):
        for k in range(SB):
            ru = rv_u[k0 + k]
            ri = rv_i[k0 + k]
            tu = pl.multiple_of((ru // 128) * 128, 128)
            ti = pl.multiple_of((ri // 128) * 128, 128)
            for db in range(4):
                pltpu.async_copy(
                    utv_hbm.at[pl.ds(db * 8, 8), pl.ds(tu, 128)],
                    ring.at[slot, k, 0, db], sem)
                pltpu.async_copy(
                    itv_hbm.at[pl.ds(db * 8, 8), pl.ds(ti, 128)],
                    ring.at[slot, k, 1, db], sem)

    def drain(sem):
        for _ in range(SB * 8):
            pltpu.make_async_copy(
                utv_hbm.at[pl.ds(0, 8), pl.ds(0, 128)],
                ring.at[0, 0, 0, 0], sem).wait()

    def compute(rv_u, rv_i, k0, slot, nb, j):
        for k in range(SB):
            lu = jnp.full((LANES,), rv_u[k0 + k] % 128, jnp.int32)
            li = jnp.full((LANES,), rv_i[k0 + k] % 128, jnp.int32)
            slot_v = jnp.full((LANES,), slot, jnp.int32)
            k_v = jnp.full((LANES,), k, jnp.int32)
            t0 = jnp.zeros((LANES,), jnp.int32)
            t1 = jnp.full((LANES,), 1, jnp.int32)
            u_lo = plsc.load_gather(ring, [slot_v, k_v, t0, db_lo, d8, lu])
            u_hi = plsc.load_gather(ring, [slot_v, k_v, t0, db_hi, d8, lu])
            i_lo = plsc.load_gather(ring, [slot_v, k_v, t1, db_lo, d8, li])
            i_hi = plsc.load_gather(ring, [slot_v, k_v, t1, db_hi, d8, li])
            acc = u_lo * i_lo + u_hi * i_hi
            tot = jnp.cumsum(acc)
            pos = nb * SUPER + j * SB + k
            plsc.store_scatter(out_v, [jnp.full((LANES,), pos, jnp.int32)],
                               tot, mask=is_last)

    rv_u0 = idx_u[pl.ds(0, LANES)]
    rv_i0 = idx_i[pl.ds(0, LANES)]
    for sb in range(DEPTH):
        issue(rv_u0, rv_i0, sb * SB, sb, sems[sb])

    def super_batch(nb, carry):
        off = pl.multiple_of(nb * SUPER, SUPER)
        rv_u = idx_u[pl.ds(off, LANES)]
        rv_i = idx_i[pl.ds(off, LANES)]
        off_n = pl.multiple_of(nb * SUPER + SUPER, SUPER)
        rv_un = idx_u[pl.ds(off_n, LANES)]
        rv_in = idx_i[pl.ds(off_n, LANES)]
        for j in range(JPS):
            slot = j % NSLOT
            jn = j + DEPTH
            slot_n = jn % NSLOT
            if jn < JPS:
                issue(rv_u, rv_i, jn * SB, slot_n, sems[slot_n])
            else:
                issue(rv_un, rv_in, (jn - JPS) * SB, slot_n, sems[slot_n])
            drain(sems[slot])
            compute(rv_u, rv_i, j * SB, slot, nb, j)
        return carry

    lax.fori_loop(0, NSUPER, super_batch, 0)
    for sb in range(DEPTH):
        drain(sems[sb % NSLOT])

    pltpu.sync_copy(out_v, out_hbm.at[pl.ds(base, BPW)])


def kernel(users, items, user_table, item_table):
    return _neumf_sc(users.astype(jnp.int32), items.astype(jnp.int32),
                     user_table.T, item_table.T)


# speedup vs baseline: 1.1108x; 1.0051x over previous
import functools

import jax
import jax.numpy as jnp
from jax import lax
from jax.experimental import pallas as pl
from jax.experimental.pallas import tpu as pltpu
from jax.experimental.pallas import tpu_sc as plsc

BATCH = 16384
NROWS = 1000000
D = 32
LANES = 16
NC = 2
NS = 16
NW = NC * NS
BPW = BATCH // NW
SB = 1
NSLOT = 8
DEPTH = 6
SUPER = 16
NSUPER = BPW // SUPER
JPS = SUPER // SB


@functools.partial(
    pl.kernel,
    out_type=jax.ShapeDtypeStruct((BATCH,), jnp.float32),
    mesh=plsc.VectorSubcoreMesh(core_axis_name="c", subcore_axis_name="s"),
    compiler_params=pltpu.CompilerParams(
        needs_layout_passes=False, disable_bounds_checks=True
    ),
    scratch_types=[
        pltpu.VMEM((BPW + SUPER,), jnp.int32),
        pltpu.VMEM((BPW + SUPER,), jnp.int32),
        pltpu.VMEM((NSLOT, SB, 2, 4, 8, 128), jnp.float32),
        pltpu.VMEM((BPW,), jnp.float32),
        pltpu.SemaphoreType.DMA,
        pltpu.SemaphoreType.DMA,
        pltpu.SemaphoreType.DMA,
        pltpu.SemaphoreType.DMA,
        pltpu.SemaphoreType.DMA,
        pltpu.SemaphoreType.DMA,
        pltpu.SemaphoreType.DMA,
        pltpu.SemaphoreType.DMA,
    ],
)
def _neumf_sc(users_hbm, items_hbm, utv_hbm, itv_hbm, out_hbm,
              idx_u, idx_i, ring, out_v,
              sem0, sem1, sem2, sem3, sem4, sem5, sem6, sem7):
    sems = (sem0, sem1, sem2, sem3, sem4, sem5, sem6, sem7)
    wid = lax.axis_index("s") * NC + lax.axis_index("c")
    base = wid * BPW

    pltpu.sync_copy(users_hbm.at[pl.ds(base, BPW)], idx_u.at[pl.ds(0, BPW)])
    pltpu.sync_copy(items_hbm.at[pl.ds(base, BPW)], idx_i.at[pl.ds(0, BPW)])
    idx_u[pl.ds(BPW, SUPER)] = jnp.zeros((SUPER,), jnp.int32)
    idx_i[pl.ds(BPW, SUPER)] = jnp.zeros((SUPER,), jnp.int32)

    lanes = lax.iota(jnp.int32, LANES)
    db_lo = lanes // 8
    db_hi = db_lo + 2
    d8 = lanes % 8
    is_last = lanes == LANES - 1

    utv4 = utv_hbm.reshape(4, 8, NROWS)
    itv4 = itv_hbm.reshape(4, 8, NROWS)

    def issue(rv_u, rv_i, k0, slot, sem):
        for k in range(SB):
            ru = rv_u[k0 + k]
            ri = rv_i[k0 + k]
            tu = pl.multiple_of((ru // 128) * 128, 128)
            ti = pl.multiple_of((ri // 128) * 128, 128)
            pltpu.async_copy(
                utv4.at[:, :, pl.ds(tu, 128)], ring.at[slot, k, 0], sem)
            pltpu.async_copy(
                itv4.at[:, :, pl.ds(ti, 128)], ring.at[slot, k, 1], sem)

    def drain(sem):
        for _ in range(SB * 2):
            pltpu.make_async_copy(
                utv4.at[:, :, pl.ds(0, 128)], ring.at[0, 0, 0], sem).wait()

    def compute(rv_u, rv_i, k0, slot, nb, j):
        for k in range(SB):
            lu = jnp.full((LANES,), rv_u[k0 + k] % 128, jnp.int32)
            li = jnp.full((LANES,), rv_i[k0 + k] % 128, jnp.int32)
            slot_v = jnp.full((LANES,), slot, jnp.int32)
            k_v = jnp.full((LANES,), k, jnp.int32)
            t0 = jnp.zeros((LANES,), jnp.int32)
            t1 = jnp.full((LANES,), 1, jnp.int32)
            u_lo = plsc.load_gather(ring, [slot_v, k_v, t0, db_lo, d8, lu])
            u_hi = plsc.load_gather(ring, [slot_v, k_v, t0, db_hi, d8, lu])
            i_lo = plsc.load_gather(ring, [slot_v, k_v, t1, db_lo, d8, li])
            i_hi = plsc.load_gather(ring, [slot_v, k_v, t1, db_hi, d8, li])
            acc = u_lo * i_lo + u_hi * i_hi
            tot = jnp.cumsum(acc)
            pos = nb * SUPER + j * SB + k
            plsc.store_scatter(out_v, [jnp.full((LANES,), pos, jnp.int32)],
                               tot, mask=is_last)

    rv_u0 = idx_u[pl.ds(0, LANES)]
    rv_i0 = idx_i[pl.ds(0, LANES)]
    for sb in range(DEPTH):
        issue(rv_u0, rv_i0, sb * SB, sb, sems[sb])

    def super_batch(nb, carry):
        off = pl.multiple_of(nb * SUPER, SUPER)
        rv_u = idx_u[pl.ds(off, LANES)]
        rv_i = idx_i[pl.ds(off, LANES)]
        off_n = pl.multiple_of(nb * SUPER + SUPER, SUPER)
        rv_un = idx_u[pl.ds(off_n, LANES)]
        rv_in = idx_i[pl.ds(off_n, LANES)]
        for j in range(JPS):
            slot = j % NSLOT
            jn = j + DEPTH
            slot_n = jn % NSLOT
            if jn < JPS:
                issue(rv_u, rv_i, jn * SB, slot_n, sems[slot_n])
            else:
                issue(rv_un, rv_in, (jn - JPS) * SB, slot_n, sems[slot_n])
            drain(sems[slot])
            compute(rv_u, rv_i, j * SB, slot, nb, j)
        return carry

    lax.fori_loop(0, NSUPER, super_batch, 0)
    for sb in range(DEPTH):
        drain(sems[sb % NSLOT])

    pltpu.sync_copy(out_v, out_hbm.at[pl.ds(base, BPW)])


def kernel(users, items, user_table, item_table):
    return _neumf_sc(users.astype(jnp.int32), items.astype(jnp.int32),
                     user_table.T, item_table.T)
